# Initial kernel scaffold; baseline (speedup 1.0000x reference)
import jax
import jax.numpy as jnp
from jax.experimental import pallas as pl


def kernel(features, ind_p1, ind_p2, W1, b1):
    raise NotImplementedError("write your pallas kernel here")


# speedup vs baseline: 5.3433x; 5.3433x over previous
import functools

import jax
import jax.numpy as jnp
from jax import lax
from jax.experimental import pallas as pl
from jax.experimental.pallas import tpu as pltpu
from jax.experimental.pallas import tpu_sc as plsc

NC = 2
NS = 16
L = 16
K = 80


def _ab_body(f_ref, wa_ref, wb_ref, b_ref, a_ref, bb_ref):
    f = f_ref[...]
    wb = wb_ref[...]
    a_ref[...] = (
        jnp.dot(f, wa_ref[...] - wb, preferred_element_type=jnp.float32)
        + b_ref[...]
    )
    bb_ref[...] = jnp.dot(f, wb, preferred_element_type=jnp.float32)


def _combine_body(f_ref, p_ref, o_ref):
    o_ref[...] = jnp.maximum(f_ref[...] + p_ref[0] + p_ref[1], 0.0)


def _edge_body(n, epw, a_hbm, b_hbm, p1_hbm, p2_hbm, out_hbm,
               idx1, idx2, rows, acc, sem):
    c = lax.axis_index("c")
    s = lax.axis_index("s")
    w = c * NS + s
    nchunks = epw // K

    @pl.loop(0, K)
    def _zero_rows(e):
        for j in range(D_VREGS):
            rows[e, pl.ds(j * L, L)] = jnp.zeros((L,), jnp.float32)

    zchunks = n // K
    ziters = (zchunks + NS - 1) // NS

    @pl.loop(0, ziters)
    def _zero_acc(i):
        chunk = i * NS + s

        @pl.when(chunk < zchunks)
        def _():
            pltpu.sync_copy(rows, acc.at[pl.ds(chunk * K, K)])

    plsc.subcore_barrier()

    base = w * epw

    @pl.loop(0, nchunks)
    def _chunk(i):
        off = base + i * K
        pltpu.sync_copy(p1_hbm.at[pl.ds(off, K)], idx1)
        pltpu.sync_copy(p2_hbm.at[pl.ds(off, K)], idx2)
        pltpu.async_copy(a_hbm.at[idx1], rows, sem).wait()
        pltpu.async_copy(b_hbm.at[idx2], rows, sem, add=True).wait()

        @pl.loop(0, K)
        def _relu(e):
            for j in range(D_VREGS):
                sl = pl.ds(j * L, L)
                rows[e, sl] = jnp.maximum(rows[e, sl], 0.0)

        pltpu.sync_copy(rows, acc.at[idx1], add=True)

    plsc.subcore_barrier()

    @pl.loop(0, ziters)
    def _writeout(i):
        chunk = i * NS + s

        @pl.when(chunk < zchunks)
        def _():
            pltpu.sync_copy(acc.at[pl.ds(chunk * K, K)],
                            out_hbm.at[c, pl.ds(chunk * K, K)])


D_VREGS = 8


def kernel(features, ind_p1, ind_p2, W1, b1):
    n, d = features.shape
    (e,) = ind_p1.shape
    assert d == D_VREGS * L
    nw = NC * NS
    epw = e // nw
    assert epw * nw == e and epw % K == 0 and n % K == 0 and n % NS == 0

    wa = W1[:d]
    wb = W1[d:]
    b2d = jnp.reshape(b1, (1, d))

    rows_blk = 1000
    grid = n // rows_blk
    a_mat, b_mat = pl.pallas_call(
        _ab_body,
        grid=(grid,),
        in_specs=[
            pl.BlockSpec((rows_blk, d), lambda i: (i, 0)),
            pl.BlockSpec((d, d), lambda i: (0, 0)),
            pl.BlockSpec((d, d), lambda i: (0, 0)),
            pl.BlockSpec((1, d), lambda i: (0, 0)),
        ],
        out_specs=[
            pl.BlockSpec((rows_blk, d), lambda i: (i, 0)),
            pl.BlockSpec((rows_blk, d), lambda i: (i, 0)),
        ],
        out_shape=[
            jax.ShapeDtypeStruct((n, d), jnp.float32),
            jax.ShapeDtypeStruct((n, d), jnp.float32),
        ],
    )(features, wa, wb, b2d)

    mesh = plsc.VectorSubcoreMesh(core_axis_name="c", subcore_axis_name="s")
    partials = pl.kernel(
        functools.partial(_edge_body, n, epw),
        out_type=jax.ShapeDtypeStruct((NC, n, d), jnp.float32),
        mesh=mesh,
        scratch_types=[
            pltpu.VMEM((K,), jnp.int32),
            pltpu.VMEM((K,), jnp.int32),
            pltpu.VMEM((K, d), jnp.float32),
            pltpu.VMEM_SHARED((n, d), jnp.float32),
            pltpu.SemaphoreType.DMA,
        ],
    )(a_mat, b_mat, ind_p1, ind_p2)

    out = pl.pallas_call(
        _combine_body,
        grid=(grid,),
        in_specs=[
            pl.BlockSpec((rows_blk, d), lambda i: (i, 0)),
            pl.BlockSpec((NC, rows_blk, d), lambda i: (0, i, 0)),
        ],
        out_specs=pl.BlockSpec((rows_blk, d), lambda i: (i, 0)),
        out_shape=jax.ShapeDtypeStruct((n, d), jnp.float32),
    )(features, partials)
    return out
